# baseline (device time: 109362 ns/iter reference)
import jax
import jax.numpy as jnp
from jax import lax
from jax.experimental import pallas as pl
from jax.experimental.pallas import tpu as pltpu

T = 2048
D = 4096
V_SHARD = 8192
V_HALF = V_SHARD // 2
V_TILE = 512
N_TILES = V_HALF // V_TILE


def _stats_body(y_sref, x_ref, w_ref, labels_ref, stats_ref, eacc_ref, lacc_ref):
    i = pl.program_id(0)
    my_x = lax.axis_index("x")
    my_y = lax.axis_index("y")

    logits = jnp.dot(
        x_ref[:, :], w_ref[:, :], preferred_element_type=jnp.float32
    )

    labs_shift = labels_ref[:, :] - (my_x * V_SHARD + my_y * V_HALF + i * V_TILE)
    cols = lax.broadcasted_iota(jnp.int16, (T, V_TILE), 1)
    mask = cols == labs_shift.astype(jnp.int16)

    @pl.when(i == 0)
    def _():
        eacc_ref[:, :] = jnp.exp(logits)
        lacc_ref[:, :] = jnp.where(mask, logits, 0.0)

    @pl.when(i != 0)
    def _():
        eacc_ref[:, :] += jnp.exp(logits)
        lacc_ref[:, :] = jnp.where(mask, logits, lacc_ref[:, :])

    @pl.when(i == N_TILES - 1)
    def _():
        stats_ref[:, 0:1] = jnp.sum(eacc_ref[:, :], axis=1, keepdims=True)
        stats_ref[:, 1:2] = jnp.sum(lacc_ref[:, :], axis=1, keepdims=True)


def _combine_body(stats_ref, out_ref, comm_ref, send_sems, recv_sems):
    my_x = lax.axis_index("x")
    my_y = lax.axis_index("y")
    peers = [(1 - my_x, my_y), (my_x, 1 - my_y), (1 - my_x, 1 - my_y)]

    barrier = pltpu.get_barrier_semaphore()
    for p in peers:
        pl.semaphore_signal(
            barrier, inc=1, device_id=p, device_id_type=pl.DeviceIdType.MESH
        )
    pl.semaphore_wait(barrier, 3)

    rdmas = []
    for k, p in enumerate(peers):
        rdma = pltpu.make_async_remote_copy(
            src_ref=stats_ref,
            dst_ref=comm_ref.at[k],
            send_sem=send_sems.at[k],
            recv_sem=recv_sems.at[k],
            device_id=p,
            device_id_type=pl.DeviceIdType.MESH,
        )
        rdma.start()
        rdmas.append(rdma)
    for rdma in rdmas:
        rdma.wait()

    tot = (
        stats_ref[:, :] + comm_ref[0, :, :] + comm_ref[1, :, :] + comm_ref[2, :, :]
    )
    out_ref[:, :] = jnp.log(tot[0:1, :]) - tot[1:2, :]


def kernel(x, W, labels):
    labels2 = labels.reshape(T, 1).astype(jnp.int32)
    my_y = lax.axis_index("y").reshape(1).astype(jnp.int32)

    stats = pl.pallas_call(
        _stats_body,
        grid_spec=pltpu.PrefetchScalarGridSpec(
            num_scalar_prefetch=1,
            grid=(N_TILES,),
            in_specs=[
                pl.BlockSpec((T, D), lambda i, y: (0, 0)),
                pl.BlockSpec((D, V_TILE), lambda i, y: (0, y[0] * N_TILES + i)),
                pl.BlockSpec((T, 1), lambda i, y: (0, 0)),
            ],
            out_specs=pl.BlockSpec((T, 2), lambda i, y: (0, 0)),
            scratch_shapes=[
                pltpu.VMEM((T, V_TILE), jnp.float32),
                pltpu.VMEM((T, V_TILE), jnp.float32),
            ],
        ),
        out_shape=jax.ShapeDtypeStruct((T, 2), jnp.float32),
        compiler_params=pltpu.CompilerParams(
            dimension_semantics=("arbitrary",),
            vmem_limit_bytes=100 * 1024 * 1024,
        ),
    )(my_y, x, W, labels2)

    stats_t = stats.T

    nll = pl.pallas_call(
        _combine_body,
        in_specs=[pl.BlockSpec(memory_space=pltpu.VMEM)],
        out_specs=pl.BlockSpec(memory_space=pltpu.VMEM),
        out_shape=jax.ShapeDtypeStruct((1, T), jnp.float32),
        scratch_shapes=[
            pltpu.VMEM((3, 2, T), jnp.float32),
            pltpu.SemaphoreType.DMA((3,)),
            pltpu.SemaphoreType.DMA((3,)),
        ],
        compiler_params=pltpu.CompilerParams(collective_id=0),
    )(stats_t)

    return nll[0]


# device time: 107445 ns/iter; 1.0178x vs baseline; 1.0178x over previous
import jax
import jax.numpy as jnp
from jax import lax
from jax.experimental import pallas as pl
from jax.experimental.pallas import tpu as pltpu

T = 2048
D = 4096
V_SHARD = 8192
V_HALF = V_SHARD // 2
V_TILE = 512
N_TILES = V_HALF // V_TILE


def _stats_body(y_sref, x_ref, w_ref, labels_ref, stats_ref, eacc_ref, lacc_ref):
    i = pl.program_id(0)
    my_x = lax.axis_index("x")
    my_y = lax.axis_index("y")

    logits = jnp.dot(
        x_ref[:, :], w_ref[:, :], preferred_element_type=jnp.float32
    )

    labs_shift = labels_ref[:, :] - (my_x * V_SHARD + my_y * V_HALF + i * V_TILE)
    cols = lax.broadcasted_iota(jnp.int16, (T, V_TILE), 1)
    mask = cols == labs_shift.astype(jnp.int16)

    e = jnp.exp(logits)
    ec = e[:, 0:128] + e[:, 128:256] + e[:, 256:384] + e[:, 384:512]
    lm = jnp.where(mask, logits, 0.0)
    lc = lm[:, 0:128] + lm[:, 128:256] + lm[:, 256:384] + lm[:, 384:512]

    @pl.when(i == 0)
    def _():
        eacc_ref[:, :] = ec
        lacc_ref[:, :] = lc

    @pl.when(i != 0)
    def _():
        eacc_ref[:, :] += ec
        lacc_ref[:, :] += lc

    @pl.when(i == N_TILES - 1)
    def _():
        stats_ref[:, 0:1] = jnp.sum(eacc_ref[:, :], axis=1, keepdims=True)
        stats_ref[:, 1:2] = jnp.sum(lacc_ref[:, :], axis=1, keepdims=True)


def _combine_body(stats_ref, out_ref, comm_ref, send_sems, recv_sems):
    my_x = lax.axis_index("x")
    my_y = lax.axis_index("y")
    peers = [(1 - my_x, my_y), (my_x, 1 - my_y), (1 - my_x, 1 - my_y)]

    barrier = pltpu.get_barrier_semaphore()
    for p in peers:
        pl.semaphore_signal(
            barrier, inc=1, device_id=p, device_id_type=pl.DeviceIdType.MESH
        )
    pl.semaphore_wait(barrier, 3)

    rdmas = []
    for k, p in enumerate(peers):
        rdma = pltpu.make_async_remote_copy(
            src_ref=stats_ref,
            dst_ref=comm_ref.at[k],
            send_sem=send_sems.at[k],
            recv_sem=recv_sems.at[k],
            device_id=p,
            device_id_type=pl.DeviceIdType.MESH,
        )
        rdma.start()
        rdmas.append(rdma)
    for rdma in rdmas:
        rdma.wait()

    tot = (
        stats_ref[:, :] + comm_ref[0, :, :] + comm_ref[1, :, :] + comm_ref[2, :, :]
    )
    out_ref[:, :] = jnp.log(tot[0:1, :]) - tot[1:2, :]


def kernel(x, W, labels):
    labels2 = labels.reshape(T, 1).astype(jnp.int32)
    my_y = lax.axis_index("y").reshape(1).astype(jnp.int32)

    stats = pl.pallas_call(
        _stats_body,
        grid_spec=pltpu.PrefetchScalarGridSpec(
            num_scalar_prefetch=1,
            grid=(N_TILES,),
            in_specs=[
                pl.BlockSpec((T, D), lambda i, y: (0, 0)),
                pl.BlockSpec((D, V_TILE), lambda i, y: (0, y[0] * N_TILES + i)),
                pl.BlockSpec((T, 1), lambda i, y: (0, 0)),
            ],
            out_specs=pl.BlockSpec((T, 2), lambda i, y: (0, 0)),
            scratch_shapes=[
                pltpu.VMEM((T, 128), jnp.float32),
                pltpu.VMEM((T, 128), jnp.float32),
            ],
        ),
        out_shape=jax.ShapeDtypeStruct((T, 2), jnp.float32),
        compiler_params=pltpu.CompilerParams(
            dimension_semantics=("arbitrary",),
            vmem_limit_bytes=100 * 1024 * 1024,
        ),
    )(my_y, x, W, labels2)

    stats_t = stats.T

    nll = pl.pallas_call(
        _combine_body,
        in_specs=[pl.BlockSpec(memory_space=pltpu.VMEM)],
        out_specs=pl.BlockSpec(memory_space=pltpu.VMEM),
        out_shape=jax.ShapeDtypeStruct((1, T), jnp.float32),
        scratch_shapes=[
            pltpu.VMEM((3, 2, T), jnp.float32),
            pltpu.SemaphoreType.DMA((3,)),
            pltpu.SemaphoreType.DMA((3,)),
        ],
        compiler_params=pltpu.CompilerParams(collective_id=0),
    )(stats_t)

    return nll[0]


# device time: 103502 ns/iter; 1.0566x vs baseline; 1.0381x over previous
import jax
import jax.numpy as jnp
from jax import lax
from jax.experimental import pallas as pl
from jax.experimental.pallas import tpu as pltpu

T = 2048
D = 4096
V_SHARD = 8192
V_HALF = V_SHARD // 2
V_TILE = 512
N_TILES = V_HALF // V_TILE


def _stats_body(y_sref, x_ref, w_ref, labels_ref, stats_ref):
    i = pl.program_id(0)
    my_x = lax.axis_index("x")
    my_y = lax.axis_index("y")

    logits = jnp.dot(
        x_ref[:, :], w_ref[:, :], preferred_element_type=jnp.float32
    )

    labs_shift = labels_ref[:, :] - (my_x * V_SHARD + my_y * V_HALF + i * V_TILE)
    cols = lax.broadcasted_iota(jnp.int32, (T, V_TILE), 1)
    sumexp = jnp.sum(jnp.exp(logits), axis=1, keepdims=True)
    lab = jnp.sum(
        jnp.where(cols == labs_shift, logits, 0.0), axis=1, keepdims=True
    )

    @pl.when(i == 0)
    def _():
        stats_ref[:, 0:1] = sumexp
        stats_ref[:, 1:2] = lab

    @pl.when(i != 0)
    def _():
        stats_ref[:, 0:1] += sumexp
        stats_ref[:, 1:2] += lab


def _combine_body(stats_ref, out_ref, comm_ref, send_sems, recv_sems):
    my_x = lax.axis_index("x")
    my_y = lax.axis_index("y")
    peers = [(1 - my_x, my_y), (my_x, 1 - my_y), (1 - my_x, 1 - my_y)]

    barrier = pltpu.get_barrier_semaphore()
    for p in peers:
        pl.semaphore_signal(
            barrier, inc=1, device_id=p, device_id_type=pl.DeviceIdType.MESH
        )
    pl.semaphore_wait(barrier, 3)

    rdmas = []
    for k, p in enumerate(peers):
        rdma = pltpu.make_async_remote_copy(
            src_ref=stats_ref,
            dst_ref=comm_ref.at[k],
            send_sem=send_sems.at[k],
            recv_sem=recv_sems.at[k],
            device_id=p,
            device_id_type=pl.DeviceIdType.MESH,
        )
        rdma.start()
        rdmas.append(rdma)
    for rdma in rdmas:
        rdma.wait()

    tot = (
        stats_ref[:, :] + comm_ref[0, :, :] + comm_ref[1, :, :] + comm_ref[2, :, :]
    )
    out_ref[:, :] = jnp.log(tot[0:1, :]) - tot[1:2, :]


def kernel(x, W, labels):
    labels2 = labels.reshape(T, 1).astype(jnp.int32)
    my_y = lax.axis_index("y").reshape(1).astype(jnp.int32)

    stats = pl.pallas_call(
        _stats_body,
        grid_spec=pltpu.PrefetchScalarGridSpec(
            num_scalar_prefetch=1,
            grid=(N_TILES,),
            in_specs=[
                pl.BlockSpec((T, D), lambda i, y: (0, 0)),
                pl.BlockSpec((D, V_TILE), lambda i, y: (0, y[0] * N_TILES + i)),
                pl.BlockSpec((T, 1), lambda i, y: (0, 0)),
            ],
            out_specs=pl.BlockSpec((T, 2), lambda i, y: (0, 0)),
        ),
        out_shape=jax.ShapeDtypeStruct((T, 2), jnp.float32),
        compiler_params=pltpu.CompilerParams(
            dimension_semantics=("arbitrary",),
            vmem_limit_bytes=100 * 1024 * 1024,
        ),
    )(my_y, x, W, labels2)

    stats_t = stats.T

    nll = pl.pallas_call(
        _combine_body,
        in_specs=[pl.BlockSpec(memory_space=pltpu.VMEM)],
        out_specs=pl.BlockSpec(memory_space=pltpu.VMEM),
        out_shape=jax.ShapeDtypeStruct((1, T), jnp.float32),
        scratch_shapes=[
            pltpu.VMEM((3, 2, T), jnp.float32),
            pltpu.SemaphoreType.DMA((3,)),
            pltpu.SemaphoreType.DMA((3,)),
        ],
        compiler_params=pltpu.CompilerParams(collective_id=0),
    )(stats_t)

    return nll[0]
